# Initial kernel scaffold; baseline (speedup 1.0000x reference)
import jax
import jax.numpy as jnp
from jax.experimental import pallas as pl


def kernel(x, edge_index, Q1, bq1, W1, bw1, Q2, bq2, W2, bw2, Gw, Gb, g):
    raise NotImplementedError("write your pallas kernel here")


# speedup vs baseline: 1.5324x; 1.5324x over previous
import jax
import jax.numpy as jnp
from jax import lax
from jax.experimental import pallas as pl
from jax.experimental.pallas import tpu as pltpu
from jax.experimental.pallas import tpu_sc as plsc

N = 10000
D = 128
E = 320000
DEGW = 16
NC, NS = 2, 16
L = 16
N2 = 10240
SPLITS = 4
NSUB = SPLITS // NC
QTR = N2 // SPLITS
DUMP = 128
HR = QTR + DUMP
EPT = E // NS
K = 80
NCHUNK = EPT // K
ZPT = HR // NS
RPT = QTR // NS


def _sc_mesh():
    return plsc.VectorSubcoreMesh(core_axis_name="c", subcore_axis_name="s")


def _sc_aggregate(nh, src, dst, zrows, with_deg):
    out_type = [jax.ShapeDtypeStruct((N2, D), jnp.float32)]
    scratch = [
        pltpu.VMEM((K,), jnp.int32),
        pltpu.VMEM((K,), jnp.int32),
        pltpu.VMEM((K, D), jnp.float32),
        pltpu.VMEM_SHARED((HR, D), jnp.float32),
    ]
    if with_deg:
        out_type.append(jax.ShapeDtypeStruct((N2 * DEGW,), jnp.float32))
        scratch += [
            pltpu.VMEM((K, DEGW), jnp.float32),
            pltpu.VMEM((RPT, DEGW), jnp.float32),
            pltpu.VMEM((RPT * DEGW,), jnp.float32),
            pltpu.VMEM((ZPT, DEGW), jnp.float32),
            pltpu.VMEM_SHARED((HR, DEGW), jnp.float32),
        ]
    scratch.append(pltpu.SemaphoreType.DMA)

    def body(nh_hbm, src_hbm, dst_hbm, zrows_hbm, *rest):
        if with_deg:
            (aggp, degp, idx_s, idx_d, msg, agg_sh,
             ones, d2d, dflat, zd, deg_sh, sem) = rest
        else:
            aggp, idx_s, idx_d, msg, agg_sh, sem = rest
        c = lax.axis_index("c")
        s = lax.axis_index("s")
        z0 = s * ZPT
        r0 = s * RPT
        ebase = s * EPT

        if with_deg:
            for i in range(ZPT):
                zd[i, :] = jnp.zeros((DEGW,), jnp.float32)
            for i in range(K):
                ones[i, :] = jnp.full((DEGW,), 1.0 / DEGW, jnp.float32)

        for p in range(NSUB):
            q = p * NC + c
            qlo = q * QTR

            pltpu.sync_copy(zrows_hbm.at[pl.ds(z0, ZPT)],
                            agg_sh.at[pl.ds(z0, ZPT)])
            if with_deg:
                pltpu.sync_copy(zd, deg_sh.at[pl.ds(z0, ZPT)])
            plsc.subcore_barrier()

            def step(i, carry):
                base = pl.multiple_of(ebase + i * K, 8)
                pltpu.sync_copy(src_hbm.at[pl.ds(base, K)], idx_s)
                pltpu.sync_copy(dst_hbm.at[pl.ds(base, K)], idx_d)
                for j in range(K // L):
                    v = idx_d[pl.ds(j * L, L)]
                    lv = v - qlo
                    inr = (lv >= 0) & (lv < QTR)
                    dump = QTR + (v & (DUMP - 1))
                    idx_d[pl.ds(j * L, L)] = jnp.where(inr, lv, dump)
                pltpu.async_copy(nh_hbm.at[idx_s], msg, sem).wait()
                pltpu.sync_copy(msg, agg_sh.at[idx_d], add=True)
                if with_deg:
                    pltpu.sync_copy(ones, deg_sh.at[idx_d], add=True)
                return carry
            lax.fori_loop(0, NCHUNK, step, 0)
            plsc.subcore_barrier()

            pltpu.sync_copy(agg_sh.at[pl.ds(r0, RPT)],
                            aggp.at[pl.ds(qlo + r0, RPT)])
            if with_deg:
                pltpu.sync_copy(deg_sh.at[pl.ds(r0, RPT)], d2d)

                for i in range(RPT):
                    dflat[pl.ds(i * DEGW, DEGW)] = d2d[i, :]
                off = (qlo + r0) * DEGW
                pltpu.sync_copy(dflat, degp.at[pl.ds(off, RPT * DEGW)])
            plsc.subcore_barrier()

    fn = pl.kernel(body, out_type=out_type, mesh=_sc_mesh(),
                   scratch_types=scratch,
                   compiler_params=pltpu.CompilerParams(
                       use_tc_tiling_on_sc=False))
    return fn(nh, src, dst, zrows)


_B = 2048


def _tc_nh(x, q, bq):
    def body(x_ref, q_ref, b_ref, o_ref):
        o_ref[...] = jnp.maximum(
            jnp.dot(x_ref[...], q_ref[...],
                    preferred_element_type=jnp.float32) + b_ref[...], 0.0)

    return pl.pallas_call(
        body,
        grid=(N2 // _B,),
        in_specs=[
            pl.BlockSpec((_B, D), lambda i: (i, 0)),
            pl.BlockSpec((D, D), lambda i: (0, 0)),
            pl.BlockSpec((1, D), lambda i: (0, 0)),
        ],
        out_specs=pl.BlockSpec((_B, D), lambda i: (i, 0)),
        out_shape=jax.ShapeDtypeStruct((N2, D), jnp.float32),
    )(x, q, bq.reshape(1, D))


def _tc_layer(xself, agg, degt, wa, wb, bw, q, bq, gval):
    def body(xs, a_r, dt, wa_r, wb_r, bw_r, q_r, bq_r, g_r, h_ref, o_ref):
        deg = jnp.sum(dt[...], axis=-1, keepdims=True)
        inv = 1.0 / jnp.maximum(deg, 1.0)
        aggm = a_r[...] * inv
        pre = (jnp.dot(xs[...], wa_r[...], preferred_element_type=jnp.float32)
               + jnp.dot(aggm, wb_r[...], preferred_element_type=jnp.float32)
               + bw_r[...])
        h = jnp.maximum(pre, 0.0)
        nrm = jnp.sqrt(jnp.sum(h * h, axis=-1, keepdims=True)) + 1e-6
        hn = h / nrm
        h_ref[...] = hn
        o_ref[...] = g_r[0] * jnp.maximum(
            jnp.dot(hn, q_r[...], preferred_element_type=jnp.float32)
            + bq_r[...], 0.0)

    nb = N2 // _B
    return pl.pallas_call(
        body,
        grid=(nb,),
        in_specs=[
            pl.BlockSpec((_B, D), lambda i: (i, 0)),
            pl.BlockSpec((_B, D), lambda i: (i, 0)),
            pl.BlockSpec((_B, DEGW), lambda i: (i, 0)),
            pl.BlockSpec((D, D), lambda i: (0, 0)),
            pl.BlockSpec((D, D), lambda i: (0, 0)),
            pl.BlockSpec((1, D), lambda i: (0, 0)),
            pl.BlockSpec((D, D), lambda i: (0, 0)),
            pl.BlockSpec((1, D), lambda i: (0, 0)),
            pl.BlockSpec(memory_space=pltpu.SMEM),
        ],
        out_specs=[
            pl.BlockSpec((_B, D), lambda i: (i, 0)),
            pl.BlockSpec((_B, D), lambda i: (i, 0)),
        ],
        out_shape=[
            jax.ShapeDtypeStruct((N2, D), jnp.float32),
            jax.ShapeDtypeStruct((N2, D), jnp.float32),
        ],
    )(xself, agg, degt, wa, wb, bw.reshape(1, D), q, bq.reshape(1, D), gval)


def kernel(x, edge_index, Q1, bq1, W1, bw1, Q2, bq2, W2, bw2, Gw, Gb, g):
    src = edge_index[0].astype(jnp.int32)
    dst = edge_index[1].astype(jnp.int32)
    x2 = jnp.pad(x, ((0, N2 - N), (0, 0)))
    zrows = jnp.zeros((HR, D), jnp.float32)

    nh1 = _tc_nh(x2, Q1, bq1)
    agg1, degp = _sc_aggregate(nh1, src, dst, zrows, with_deg=True)
    degt = degp.reshape(N2, DEGW)
    h1, nh2 = _tc_layer(x2, agg1, degt, W1[:D], W1[D:], bw1, Q2, bq2,
                        jnp.ones((1,), jnp.float32))
    agg2 = _sc_aggregate(nh2, src, dst, zrows, with_deg=False)[0]
    h2, out = _tc_layer(h1, agg2, degt, W2[:D], W2[D:], bw2, Gw, Gb, g)
    del h2
    return out[:N]


# speedup vs baseline: 4.0384x; 2.6353x over previous
import jax
import jax.numpy as jnp
from jax import lax
from jax.experimental import pallas as pl
from jax.experimental.pallas import tpu as pltpu
from jax.experimental.pallas import tpu_sc as plsc

N = 10000
D = 128
E = 320000
DEGW = 16
NC, NS = 2, 16
L = 16
N2 = 10240
SPLITS = 4
NSUB = SPLITS // NC
QTR = N2 // SPLITS
DUMP = 128
HR = QTR + DUMP
EPT = E // NS
K = 80
NCHUNK = EPT // K
ZPT = HR // NS
RPT = QTR // NS


def _sc_mesh():
    return plsc.VectorSubcoreMesh(core_axis_name="c", subcore_axis_name="s")


def _sc_aggregate(nh, src, dst, zrows, with_deg):
    out_type = [jax.ShapeDtypeStruct((N2, D), jnp.float32)]
    scratch = [
        pltpu.VMEM((EPT,), jnp.int32),
        pltpu.VMEM((EPT,), jnp.int32),
        pltpu.VMEM((K,), jnp.int32),
        pltpu.VMEM((K,), jnp.int32),
        pltpu.VMEM((K,), jnp.int32),
        pltpu.VMEM((K,), jnp.int32),
        pltpu.VMEM((K, D), jnp.float32),
        pltpu.VMEM((K, D), jnp.float32),
        pltpu.VMEM_SHARED((HR, D), jnp.float32),
        pltpu.SemaphoreType.DMA,
    ]
    if with_deg:
        out_type.append(jax.ShapeDtypeStruct((N2 * DEGW,), jnp.float32))
        scratch += [
            pltpu.VMEM((K, DEGW), jnp.float32),
            pltpu.VMEM((RPT, DEGW), jnp.float32),
            pltpu.VMEM((RPT * DEGW,), jnp.float32),
            pltpu.VMEM((ZPT, DEGW), jnp.float32),
            pltpu.VMEM_SHARED((HR, DEGW), jnp.float32),
        ]
    scratch.append(pltpu.SemaphoreType.DMA)

    def body(nh_hbm, src_hbm, dst_hbm, zrows_hbm, *rest):
        if with_deg:
            (aggp, degp, src_big, dst_big, is0, is1, id0, id1, m0, m1,
             agg_sh, sem0, ones, d2d, dflat, zd, deg_sh, sem1) = rest
        else:
            (aggp, src_big, dst_big, is0, is1, id0, id1, m0, m1,
             agg_sh, sem0, sem1) = rest
        isb, idb, msgb, semb = [is0, is1], [id0, id1], [m0, m1], [sem0, sem1]
        c = lax.axis_index("c")
        s = lax.axis_index("s")
        z0 = s * ZPT
        r0 = s * RPT
        ebase = s * EPT

        if with_deg:
            for i in range(ZPT):
                zd[i, :] = jnp.zeros((DEGW,), jnp.float32)
            for i in range(K):
                ones[i, :] = jnp.full((DEGW,), 1.0 / DEGW, jnp.float32)

        pltpu.sync_copy(src_hbm.at[pl.ds(ebase, EPT)], src_big)
        pltpu.sync_copy(dst_hbm.at[pl.ds(ebase, EPT)], dst_big)

        for p in range(NSUB):
            q = p * NC + c
            qlo = q * QTR

            pltpu.sync_copy(zrows_hbm.at[pl.ds(z0, ZPT)],
                            agg_sh.at[pl.ds(z0, ZPT)])
            if with_deg:
                pltpu.sync_copy(zd, deg_sh.at[pl.ds(z0, ZPT)])
            plsc.subcore_barrier()

            def prep(ci, b):
                for j in range(K // L):
                    o = ci * K + j * L
                    isb[b][pl.ds(j * L, L)] = src_big[pl.ds(o, L)]
                    v = dst_big[pl.ds(o, L)]
                    lv = v - qlo
                    inr = (lv >= 0) & (lv < QTR)
                    dump = QTR + (v & (DUMP - 1))
                    idb[b][pl.ds(j * L, L)] = jnp.where(inr, lv, dump)

            def fire(b):
                pltpu.make_async_copy(nh_hbm.at[isb[b]], msgb[b],
                                      semb[b]).start()

            def drain(b):
                pltpu.make_async_copy(nh_hbm.at[isb[b]], msgb[b],
                                      semb[b]).wait()

            def scat(b):
                pltpu.sync_copy(msgb[b], agg_sh.at[idb[b]], add=True)
                if with_deg:
                    pltpu.sync_copy(ones, deg_sh.at[idb[b]], add=True)

            prep(0, 0)
            fire(0)

            def group(g, carry):
                prep(2 * g + 1, 1)
                fire(1)
                drain(0)
                scat(0)
                prep(jnp.minimum(2 * g + 2, NCHUNK - 1), 0)
                fire(0)
                drain(1)
                scat(1)
                return carry
            lax.fori_loop(0, NCHUNK // 2, group, 0)
            drain(0)
            plsc.subcore_barrier()

            pltpu.sync_copy(agg_sh.at[pl.ds(r0, RPT)],
                            aggp.at[pl.ds(qlo + r0, RPT)])
            if with_deg:
                pltpu.sync_copy(deg_sh.at[pl.ds(r0, RPT)], d2d)

                for i in range(RPT):
                    dflat[pl.ds(i * DEGW, DEGW)] = d2d[i, :]
                off = (qlo + r0) * DEGW
                pltpu.sync_copy(dflat, degp.at[pl.ds(off, RPT * DEGW)])
            plsc.subcore_barrier()

    fn = pl.kernel(body, out_type=out_type, mesh=_sc_mesh(),
                   scratch_types=scratch,
                   compiler_params=pltpu.CompilerParams(
                       use_tc_tiling_on_sc=False))
    return fn(nh, src, dst, zrows)


_B = 2048


def _tc_nh(x, q, bq):
    def body(x_ref, q_ref, b_ref, o_ref):
        o_ref[...] = jnp.maximum(
            jnp.dot(x_ref[...], q_ref[...],
                    preferred_element_type=jnp.float32) + b_ref[...], 0.0)

    return pl.pallas_call(
        body,
        grid=(N2 // _B,),
        in_specs=[
            pl.BlockSpec((_B, D), lambda i: (i, 0)),
            pl.BlockSpec((D, D), lambda i: (0, 0)),
            pl.BlockSpec((1, D), lambda i: (0, 0)),
        ],
        out_specs=pl.BlockSpec((_B, D), lambda i: (i, 0)),
        out_shape=jax.ShapeDtypeStruct((N2, D), jnp.float32),
    )(x, q, bq.reshape(1, D))


def _tc_layer(xself, agg, degt, wa, wb, bw, q, bq, gval):
    def body(xs, a_r, dt, wa_r, wb_r, bw_r, q_r, bq_r, g_r, h_ref, o_ref):
        deg = jnp.sum(dt[...], axis=-1, keepdims=True)
        inv = 1.0 / jnp.maximum(deg, 1.0)
        aggm = a_r[...] * inv
        pre = (jnp.dot(xs[...], wa_r[...], preferred_element_type=jnp.float32)
               + jnp.dot(aggm, wb_r[...], preferred_element_type=jnp.float32)
               + bw_r[...])
        h = jnp.maximum(pre, 0.0)
        nrm = jnp.sqrt(jnp.sum(h * h, axis=-1, keepdims=True)) + 1e-6
        hn = h / nrm
        h_ref[...] = hn
        o_ref[...] = g_r[0] * jnp.maximum(
            jnp.dot(hn, q_r[...], preferred_element_type=jnp.float32)
            + bq_r[...], 0.0)

    nb = N2 // _B
    return pl.pallas_call(
        body,
        grid=(nb,),
        in_specs=[
            pl.BlockSpec((_B, D), lambda i: (i, 0)),
            pl.BlockSpec((_B, D), lambda i: (i, 0)),
            pl.BlockSpec((_B, DEGW), lambda i: (i, 0)),
            pl.BlockSpec((D, D), lambda i: (0, 0)),
            pl.BlockSpec((D, D), lambda i: (0, 0)),
            pl.BlockSpec((1, D), lambda i: (0, 0)),
            pl.BlockSpec((D, D), lambda i: (0, 0)),
            pl.BlockSpec((1, D), lambda i: (0, 0)),
            pl.BlockSpec(memory_space=pltpu.SMEM),
        ],
        out_specs=[
            pl.BlockSpec((_B, D), lambda i: (i, 0)),
            pl.BlockSpec((_B, D), lambda i: (i, 0)),
        ],
        out_shape=[
            jax.ShapeDtypeStruct((N2, D), jnp.float32),
            jax.ShapeDtypeStruct((N2, D), jnp.float32),
        ],
    )(xself, agg, degt, wa, wb, bw.reshape(1, D), q, bq.reshape(1, D), gval)


def kernel(x, edge_index, Q1, bq1, W1, bw1, Q2, bq2, W2, bw2, Gw, Gb, g):
    src = edge_index[0].astype(jnp.int32)
    dst = edge_index[1].astype(jnp.int32)
    x2 = jnp.pad(x, ((0, N2 - N), (0, 0)))
    zrows = jnp.zeros((HR, D), jnp.float32)

    nh1 = _tc_nh(x2, Q1, bq1)
    agg1, degp = _sc_aggregate(nh1, src, dst, zrows, with_deg=True)
    degt = degp.reshape(N2, DEGW)
    h1, nh2 = _tc_layer(x2, agg1, degt, W1[:D], W1[D:], bw1, Q2, bq2,
                        jnp.ones((1,), jnp.float32))
    agg2 = _sc_aggregate(nh2, src, dst, zrows, with_deg=False)[0]
    h2, out = _tc_layer(h1, agg2, degt, W2[:D], W2[D:], bw2, Gw, Gb, g)
    del h2
    return out[:N]


# speedup vs baseline: 9.3482x; 2.3148x over previous
import jax
import jax.numpy as jnp
from jax import lax
from jax.experimental import pallas as pl
from jax.experimental.pallas import tpu as pltpu
from jax.experimental.pallas import tpu_sc as plsc

N = 10000
D = 128
E = 320000
DEGW = 16
NC, NS = 2, 16
L = 16
N2 = 10240
SPLITS = 4
NSUB = SPLITS // NC
QTR = N2 // SPLITS
DUMP = 128
HR = QTR + DUMP
EPT = E // NS
K = 80
NCHUNK = EPT // K
ZPT = HR // NS
RPT = QTR // NS
CAP = 8000


def _sc_mesh():
    return plsc.VectorSubcoreMesh(core_axis_name="c", subcore_axis_name="s")


def _sc_aggregate(nh, src, dst, zrows, with_deg):
    out_type = [jax.ShapeDtypeStruct((N2, D), jnp.float32)]
    scratch = [
        pltpu.VMEM((EPT,), jnp.int32),
        pltpu.VMEM((EPT,), jnp.int32),
        pltpu.VMEM((CAP,), jnp.int32),
        pltpu.VMEM((CAP,), jnp.int32),
        pltpu.VMEM((K,), jnp.int32),
        pltpu.VMEM((K,), jnp.int32),
        pltpu.VMEM((K,), jnp.int32),
        pltpu.VMEM((K,), jnp.int32),
        pltpu.VMEM((K, D), jnp.float32),
        pltpu.VMEM((K, D), jnp.float32),
        pltpu.VMEM_SHARED((HR, D), jnp.float32),
        pltpu.SemaphoreType.DMA,
    ]
    if with_deg:
        out_type.append(jax.ShapeDtypeStruct((N2 * DEGW,), jnp.float32))
        scratch += [
            pltpu.VMEM((K, DEGW), jnp.float32),
            pltpu.VMEM((RPT, DEGW), jnp.float32),
            pltpu.VMEM((RPT * DEGW,), jnp.float32),
            pltpu.VMEM((ZPT, DEGW), jnp.float32),
            pltpu.VMEM_SHARED((HR, DEGW), jnp.float32),
        ]
    scratch.append(pltpu.SemaphoreType.DMA)

    def body(nh_hbm, src_hbm, dst_hbm, zrows_hbm, *rest):
        if with_deg:
            (aggp, degp, src_big, dst_big, cs, cd, is0, is1, id0, id1,
             m0, m1, agg_sh, sem0, ones, d2d, dflat, zd, deg_sh,
             sem1) = rest
        else:
            (aggp, src_big, dst_big, cs, cd, is0, is1, id0, id1, m0, m1,
             agg_sh, sem0, sem1) = rest
        isb, idb, msgb, semb = [is0, is1], [id0, id1], [m0, m1], [sem0, sem1]
        c = lax.axis_index("c")
        s = lax.axis_index("s")
        z0 = s * ZPT
        r0 = s * RPT
        ebase = s * EPT

        if with_deg:
            for i in range(ZPT):
                zd[i, :] = jnp.zeros((DEGW,), jnp.float32)
            for i in range(K):
                ones[i, :] = jnp.full((DEGW,), 1.0 / DEGW, jnp.float32)

        pltpu.sync_copy(src_hbm.at[pl.ds(ebase, EPT)], src_big)
        pltpu.sync_copy(dst_hbm.at[pl.ds(ebase, EPT)], dst_big)

        for p in range(NSUB):
            q = p * NC + c
            qlo = q * QTR

            pltpu.sync_copy(zrows_hbm.at[pl.ds(z0, ZPT)],
                            agg_sh.at[pl.ds(z0, ZPT)])
            if with_deg:
                pltpu.sync_copy(zd, deg_sh.at[pl.ds(z0, ZPT)])
            plsc.subcore_barrier()

            def compact(t, ptr):
                sv = src_big[pl.ds(t * L, L)]
                lv = dst_big[pl.ds(t * L, L)] - qlo
                inr = (lv >= 0) & (lv < QTR)
                cnt = plsc.cumsum(inr.astype(jnp.int32))
                trash = (CAP - L) + lax.iota(jnp.int32, L)
                pos = jnp.where(inr, ptr + cnt - 1, trash)
                plsc.store_scatter(cs, [pos], sv)
                plsc.store_scatter(cd, [pos], jnp.where(inr, lv, 0))
                return ptr + jnp.max(cnt)
            ptr = lax.fori_loop(0, EPT // L, compact, jnp.int32(0))
            iot = lax.iota(jnp.int32, L)
            for j in range(2 * (K // L) + 1):
                cs[pl.ds(ptr + j * L, L)] = j * L + iot
                cd[pl.ds(ptr + j * L, L)] = QTR + ((j % 8) * L + iot)
            nch = (ptr + K - 1) // K
            nch = nch + (nch & 1)
            npair = nch // 2

            def prep(ci, b):
                for j in range(K // L):
                    o = ci * K + j * L
                    isb[b][pl.ds(j * L, L)] = cs[pl.ds(o, L)]
                    idb[b][pl.ds(j * L, L)] = cd[pl.ds(o, L)]

            def fire(b):
                pltpu.make_async_copy(nh_hbm.at[isb[b]], msgb[b],
                                      semb[b]).start()

            def drain(b):
                pltpu.make_async_copy(nh_hbm.at[isb[b]], msgb[b],
                                      semb[b]).wait()

            def scat(b):
                pltpu.sync_copy(msgb[b], agg_sh.at[idb[b]], add=True)
                if with_deg:
                    pltpu.sync_copy(ones, deg_sh.at[idb[b]], add=True)

            prep(0, 0)
            fire(0)

            def group(g, carry):
                prep(2 * g + 1, 1)
                fire(1)
                drain(0)
                scat(0)
                prep(jnp.minimum(2 * g + 2, nch - 1), 0)
                fire(0)
                drain(1)
                scat(1)
                return carry
            lax.fori_loop(0, npair, group, 0)
            drain(0)
            plsc.subcore_barrier()

            pltpu.sync_copy(agg_sh.at[pl.ds(r0, RPT)],
                            aggp.at[pl.ds(qlo + r0, RPT)])
            if with_deg:
                pltpu.sync_copy(deg_sh.at[pl.ds(r0, RPT)], d2d)

                for i in range(RPT):
                    dflat[pl.ds(i * DEGW, DEGW)] = d2d[i, :]
                off = (qlo + r0) * DEGW
                pltpu.sync_copy(dflat, degp.at[pl.ds(off, RPT * DEGW)])
            plsc.subcore_barrier()

    fn = pl.kernel(body, out_type=out_type, mesh=_sc_mesh(),
                   scratch_types=scratch,
                   compiler_params=pltpu.CompilerParams(
                       use_tc_tiling_on_sc=False,
                       needs_layout_passes=False))
    return fn(nh, src, dst, zrows)


_B = 2048


def _tc_nh(x, q, bq):
    def body(x_ref, q_ref, b_ref, o_ref):
        o_ref[...] = jnp.maximum(
            jnp.dot(x_ref[...], q_ref[...],
                    preferred_element_type=jnp.float32) + b_ref[...], 0.0)

    return pl.pallas_call(
        body,
        grid=(N2 // _B,),
        in_specs=[
            pl.BlockSpec((_B, D), lambda i: (i, 0)),
            pl.BlockSpec((D, D), lambda i: (0, 0)),
            pl.BlockSpec((1, D), lambda i: (0, 0)),
        ],
        out_specs=pl.BlockSpec((_B, D), lambda i: (i, 0)),
        out_shape=jax.ShapeDtypeStruct((N2, D), jnp.float32),
    )(x, q, bq.reshape(1, D))


def _tc_layer(xself, agg, degt, wa, wb, bw, q, bq, gval):
    def body(xs, a_r, dt, wa_r, wb_r, bw_r, q_r, bq_r, g_r, h_ref, o_ref):
        deg = jnp.sum(dt[...], axis=-1, keepdims=True)
        inv = 1.0 / jnp.maximum(deg, 1.0)
        aggm = a_r[...] * inv
        pre = (jnp.dot(xs[...], wa_r[...], preferred_element_type=jnp.float32)
               + jnp.dot(aggm, wb_r[...], preferred_element_type=jnp.float32)
               + bw_r[...])
        h = jnp.maximum(pre, 0.0)
        nrm = jnp.sqrt(jnp.sum(h * h, axis=-1, keepdims=True)) + 1e-6
        hn = h / nrm
        h_ref[...] = hn
        o_ref[...] = g_r[0] * jnp.maximum(
            jnp.dot(hn, q_r[...], preferred_element_type=jnp.float32)
            + bq_r[...], 0.0)

    nb = N2 // _B
    return pl.pallas_call(
        body,
        grid=(nb,),
        in_specs=[
            pl.BlockSpec((_B, D), lambda i: (i, 0)),
            pl.BlockSpec((_B, D), lambda i: (i, 0)),
            pl.BlockSpec((_B, DEGW), lambda i: (i, 0)),
            pl.BlockSpec((D, D), lambda i: (0, 0)),
            pl.BlockSpec((D, D), lambda i: (0, 0)),
            pl.BlockSpec((1, D), lambda i: (0, 0)),
            pl.BlockSpec((D, D), lambda i: (0, 0)),
            pl.BlockSpec((1, D), lambda i: (0, 0)),
            pl.BlockSpec(memory_space=pltpu.SMEM),
        ],
        out_specs=[
            pl.BlockSpec((_B, D), lambda i: (i, 0)),
            pl.BlockSpec((_B, D), lambda i: (i, 0)),
        ],
        out_shape=[
            jax.ShapeDtypeStruct((N2, D), jnp.float32),
            jax.ShapeDtypeStruct((N2, D), jnp.float32),
        ],
    )(xself, agg, degt, wa, wb, bw.reshape(1, D), q, bq.reshape(1, D), gval)


def kernel(x, edge_index, Q1, bq1, W1, bw1, Q2, bq2, W2, bw2, Gw, Gb, g):
    src = edge_index[0].astype(jnp.int32)
    dst = edge_index[1].astype(jnp.int32)
    x2 = jnp.pad(x, ((0, N2 - N), (0, 0)))
    zrows = jnp.zeros((HR, D), jnp.float32)

    nh1 = _tc_nh(x2, Q1, bq1)
    agg1, degp = _sc_aggregate(nh1, src, dst, zrows, with_deg=True)
    degt = degp.reshape(N2, DEGW)
    h1, nh2 = _tc_layer(x2, agg1, degt, W1[:D], W1[D:], bw1, Q2, bq2,
                        jnp.ones((1,), jnp.float32))
    agg2 = _sc_aggregate(nh2, src, dst, zrows, with_deg=False)[0]
    h2, out = _tc_layer(h1, agg2, degt, W2[:D], W2[D:], bw2, Gw, Gb, g)
    del h2
    return out[:N]
